# Initial kernel scaffold; baseline (speedup 1.0000x reference)
import jax
import jax.numpy as jnp
from jax.experimental import pallas as pl


def kernel(triangle01, triangle02):
    raise NotImplementedError("write your pallas kernel here")


# speedup vs baseline: 2.3171x; 2.3171x over previous
import functools

import jax
import jax.numpy as jnp
from jax import lax
from jax.experimental import pallas as pl
from jax.experimental.pallas import tpu as pltpu
from jax.experimental.pallas import tpu_sc as plsc

P = 512
PAD = 16
W = P + 2 * PAD
B = 32
L = 16
NCHUNK = W // L

_cached = {}


def _build():
    info = plsc.get_sparse_core_info()
    nc, ns = info.num_cores, info.num_subcores
    mesh = plsc.VectorSubcoreMesh(core_axis_name="c", subcore_axis_name="s")

    @functools.partial(
        pl.kernel,
        mesh=mesh,
        out_type=jax.ShapeDtypeStruct((B, P, W), jnp.float32),
        compiler_params=pltpu.CompilerParams(
            use_tc_tiling_on_sc=False, needs_layout_passes=False
        ),
        scratch_types=[
            pltpu.VMEM((P + PAD, L), jnp.float32),
            pltpu.VMEM((L, P), jnp.float32),
            pltpu.VMEM((L, W), jnp.float32),
        ],
    )
    def shear_kernel(t01, t02, out, S, T, O):
        wid = lax.axis_index("s") * nc + lax.axis_index("c")
        iota = lax.iota(jnp.int32, L)
        zf = jnp.zeros((L,), jnp.float32)
        for rr in range(P, P + PAD):
            S[rr, :] = zf

        def unit_body(uu, carry):
            b = wid
            C = uu * L
            i0 = (P - L) - C
            pltpu.sync_copy(t02.at[b, :, pl.ds(C, L)], S.at[pl.ds(0, P)])
            pltpu.sync_copy(t01.at[b, pl.ds(i0, L), :], T)

            def row_body(li, carry2):
                i = i0 + li
                cc = jnp.full((L,), (L - 1) - li, jnp.int32)
                liv = jnp.full((L,), li, jnp.int32)
                rv = (PAD + i) - iota
                cv = iota - (i + PAD + 1)
                for m in range(NCHUNK):
                    gS = plsc.load_gather(S, [jnp.maximum(rv, 0), cc])
                    gT = plsc.load_gather(T, [liv, jnp.clip(cv, 0, P - 1)])
                    val = jnp.where(rv >= 0, gS, jnp.where(cv >= P, zf, gT))
                    O[li, pl.ds(m * L, L)] = val
                    if m != NCHUNK - 1:
                        rv = rv - L
                        cv = cv + L
                return carry2

            lax.fori_loop(0, L, row_body, 0)
            pltpu.sync_copy(O, out.at[b, pl.ds(i0, L), :])
            return carry

        lax.fori_loop(0, B, unit_body, 0)

    return shear_kernel


def kernel(triangle01, triangle02):
    if "k" not in _cached:
        _cached["k"] = _build()
    return _cached["k"](triangle01, triangle02)


# speedup vs baseline: 3.1736x; 1.3696x over previous
import functools

import jax
import jax.numpy as jnp
from jax import lax
from jax.experimental import pallas as pl
from jax.experimental.pallas import tpu as pltpu
from jax.experimental.pallas import tpu_sc as plsc

P = 512
PAD = 16
W = P + 2 * PAD
B = 32
L = 16
NCHUNK = W // L
NU = P // L

_cached = {}


def _build():
    info = plsc.get_sparse_core_info()
    nc = info.num_cores
    mesh = plsc.VectorSubcoreMesh(core_axis_name="c", subcore_axis_name="s")

    scratch = [
        pltpu.VMEM((P + PAD, L), jnp.float32),
        pltpu.VMEM((P + PAD, L), jnp.float32),
        pltpu.VMEM((L, P), jnp.float32),
        pltpu.VMEM((L, P), jnp.float32),
        pltpu.VMEM((L, W), jnp.float32),
        pltpu.VMEM((L, W), jnp.float32),
    ] + [pltpu.SemaphoreType.DMA] * 6

    @functools.partial(
        pl.kernel,
        mesh=mesh,
        out_type=jax.ShapeDtypeStruct((B, P, W), jnp.float32),
        compiler_params=pltpu.CompilerParams(
            use_tc_tiling_on_sc=False, needs_layout_passes=False
        ),
        scratch_types=scratch,
    )
    def shear_kernel(t01, t02, out, S0, S1, T0, T1, O0, O1,
                     sS0, sS1, sT0, sT1, sO0, sO1):
        b = lax.axis_index("s") * nc + lax.axis_index("c")
        iota = lax.iota(jnp.int32, L)
        zf = jnp.zeros((L,), jnp.float32)
        Sb, Tb, Ob = (S0, S1), (T0, T1), (O0, O1)
        sS, sT, sO = (sS0, sS1), (sT0, sT1), (sO0, sO1)
        for Sp in Sb:
            for rr in range(P, P + PAD):
                Sp[rr, :] = zf

        def issue_in(uu, p):
            C = L * uu
            i0 = (P - L) - C
            pltpu.async_copy(t02.at[b, :, pl.ds(C, L)], Sb[p].at[pl.ds(0, P)],
                             sS[p])
            pltpu.async_copy(t01.at[b, pl.ds(i0, L), :], Tb[p], sT[p])

        def wait_in(p):
            pltpu.make_async_copy(t02.at[b, :, pl.ds(0, L)],
                                  Sb[p].at[pl.ds(0, P)], sS[p]).wait()
            pltpu.make_async_copy(t01.at[b, pl.ds(0, L), :], Tb[p],
                                  sT[p]).wait()

        def wait_out(p):
            pltpu.make_async_copy(Ob[p], out.at[b, pl.ds(0, L), :],
                                  sO[p]).wait()

        def compute_unit(uu, p):
            i0 = (P - L) - L * uu
            q = NU - 1 - uu
            S, T, O = Sb[p], Tb[p], Ob[p]

            def row_body(li, carry):
                i = i0 + li
                ccv = (L - 1) - li + iota * 0
                liv = li + iota * 0
                rv0 = (PAD + i) - iota

                def t2_four(_, c):
                    rv, om = c
                    for _ in range(4):
                        O[li, pl.ds(om, L)] = plsc.load_gather(S, [rv, ccv])
                        rv = rv - L
                        om = om + L
                    return rv, om

                na = q // 4
                lax.fori_loop(0, na, t2_four, (rv0, 0))

                def gen_body(m, carry2):
                    sm = m * L
                    rm = (PAD + i - sm) - iota
                    cm = (sm - i - PAD - 1) + iota
                    gS = plsc.load_gather(S, [jnp.maximum(rm, 0), ccv])
                    gT = plsc.load_gather(T, [liv, jnp.clip(cm, 0, P - 1)])
                    val = jnp.where(rm >= 0, gS, jnp.where(cm >= P, zf, gT))
                    O[li, pl.ds(sm, L)] = val
                    return carry2

                lax.fori_loop(4 * na, q + 2, gen_body, 0)

                n01 = NU - q - jnp.where(q == 0, 1, 0)
                nb = n01 // 4

                def t01_four(_, c):
                    cs, om = c
                    for _ in range(4):
                        O[li, pl.ds(om, L)] = T[li, pl.ds(cs, L)]
                        cs = cs + L
                        om = om + L
                    return cs, om

                lax.fori_loop(0, nb, t01_four, (L - 1 - li, (q + 2) * L))

                lax.fori_loop(q + 2 + 4 * nb, NCHUNK, gen_body, 0)
                return carry

            lax.fori_loop(0, L, row_body, 0)
            pltpu.async_copy(O, out.at[b, pl.ds(i0, L), :], sO[p])

        issue_in(0, 0)
        issue_in(1, 1)

        def pair_body(j, carry):
            for p in (0, 1):
                uu = 2 * j + p
                wait_in(p)

                @pl.when(j > 0)
                def _():
                    wait_out(p)

                compute_unit(uu, p)

                @pl.when(j < NU // 2 - 1)
                def _():
                    issue_in(uu + 2, p)
            return carry

        lax.fori_loop(0, NU // 2, pair_body, 0)
        wait_out(0)
        wait_out(1)

    return shear_kernel


def kernel(triangle01, triangle02):
    if "k" not in _cached:
        _cached["k"] = _build()
    return _cached["k"](triangle01, triangle02)


# speedup vs baseline: 3.6890x; 1.1624x over previous
import functools

import jax
import jax.numpy as jnp
from jax import lax
from jax.experimental import pallas as pl
from jax.experimental.pallas import tpu as pltpu
from jax.experimental.pallas import tpu_sc as plsc

P = 512
PAD = 16
W = P + 2 * PAD
B = 32
L = 16
NCHUNK = W // L
NU = P // L
SLABW = 128

_cached = {}


def _build():
    info = plsc.get_sparse_core_info()
    nc = info.num_cores
    mesh = plsc.VectorSubcoreMesh(core_axis_name="c", subcore_axis_name="s")

    scratch = [
        pltpu.VMEM((P, SLABW), jnp.float32),
        pltpu.VMEM((4 * L, SLABW), jnp.float32),
        pltpu.VMEM((4 * L, SLABW), jnp.float32),
        pltpu.VMEM((L, W), jnp.float32),
        pltpu.VMEM((L, W), jnp.float32),
    ] + [pltpu.SemaphoreType.DMA] * 5

    @functools.partial(
        pl.kernel,
        mesh=mesh,
        out_type=jax.ShapeDtypeStruct((B, P, W), jnp.float32),
        compiler_params=pltpu.CompilerParams(needs_layout_passes=False),
        scratch_types=scratch,
    )
    def shear_kernel(t01, t02, out, S, T0, T1, O0, O1, sS, sT0, sT1, sO0, sO1):
        b = lax.axis_index("s") * nc + lax.axis_index("c")
        iota = lax.iota(jnp.int32, L)
        zf = jnp.zeros((L,), jnp.float32)
        Tb, Ob = (T0, T1), (O0, O1)
        sT, sO = (sT0, sT1), (sO0, sO1)

        slab_rows = [P, 400, 272, 144]

        def unit_i0(g):
            return (P - SLABW) - SLABW * (g >> 3) + L * (g & 7)

        def issue_slab(j):
            for s in range(4):
                @pl.when(j == 4 * s)
                def _(s=s):
                    nr = slab_rows[s]
                    pltpu.async_copy(
                        t02.at[b, pl.ds(0, nr), pl.ds(SLABW * s, SLABW)],
                        S.at[pl.ds(0, nr)], sS)
                    pltpu.make_async_copy(
                        t02.at[b, pl.ds(0, nr), pl.ds(0, SLABW)],
                        S.at[pl.ds(0, nr)], sS).wait()

        def issue_in(g, p):
            i0 = unit_i0(g)
            for tt in range(4):
                pltpu.async_copy(
                    t01.at[b, pl.ds(i0, L), pl.ds(SLABW * tt, SLABW)],
                    Tb[p].at[pl.ds(L * tt, L)], sT[p])

        def wait_in(p):
            for tt in range(4):
                pltpu.make_async_copy(
                    t01.at[b, pl.ds(0, L), pl.ds(SLABW * tt, SLABW)],
                    Tb[p].at[pl.ds(L * tt, L)], sT[p]).wait()

        def wait_out(p):
            pltpu.make_async_copy(Ob[p], out.at[b, pl.ds(0, L), :],
                                  sO[p]).wait()

        def compute_unit(g, p):
            i0 = unit_i0(g)
            q = i0 // L
            cslab = (P - 1) - i0 - SLABW * (g >> 3)
            T, O = Tb[p], Ob[p]
            m0 = jnp.where(q == NU - 1, 1, 0)

            def row_body(li, carry):
                i = i0 + li
                ccv = (cslab - li) + iota * 0
                liv = li + iota * 0

                def gen_body(m, carry2):
                    sm = m * L
                    rm = (PAD + i - sm) - iota
                    cm = (sm - i - PAD - 1) + iota
                    gS = plsc.load_gather(S, [jnp.clip(rm, 0, P - 1), ccv])
                    cmc = jnp.clip(cm, 0, P - 1)
                    rowT = ((cmc >> 7) << 4) + liv
                    gT = plsc.load_gather(T, [rowT, cmc & (SLABW - 1)])
                    val = jnp.where(
                        rm >= 0,
                        jnp.where(rm >= P, zf, gS),
                        jnp.where(cm >= P, zf, gT))
                    O[li, pl.ds(sm, L)] = val
                    return carry2

                lax.fori_loop(0, m0, gen_body, 0)

                def t2_four(_, c):
                    rv, om = c
                    for _ in range(4):
                        O[li, pl.ds(om, L)] = plsc.load_gather(S, [rv, ccv])
                        rv = rv - L
                        om = om + L
                    return rv, om

                na = (q - m0) // 4
                rv0 = (PAD + i - L * m0) - iota
                lax.fori_loop(0, na, t2_four, (rv0, m0 * L))

                lax.fori_loop(m0 + 4 * na, q + 2, gen_body, 0)

                n01 = NU - q - jnp.where(q == 0, 1, 0)
                nb = n01 // 4

                def t01_four(_, c):
                    cv, om = c
                    for _ in range(4):
                        rowT = ((cv >> 7) << 4) + liv
                        O[li, pl.ds(om, L)] = plsc.load_gather(
                            T, [rowT, cv & (SLABW - 1)])
                        cv = cv + L
                        om = om + L
                    return cv, om

                lax.fori_loop(0, nb, t01_four,
                              ((L - 1 - li) + iota, (q + 2) * L))

                lax.fori_loop(q + 2 + 4 * nb, NCHUNK, gen_body, 0)
                return carry

            lax.fori_loop(0, L, row_body, 0)
            pltpu.async_copy(O, out.at[b, pl.ds(i0, L), :], sO[p])

        issue_in(0, 0)
        issue_in(1, 1)

        def pair_body(j, carry):
            issue_slab(j)
            for p in (0, 1):
                g = 2 * j + p
                wait_in(p)

                @pl.when(j > 0)
                def _():
                    wait_out(p)

                compute_unit(g, p)

                @pl.when(j < NU // 2 - 1)
                def _():
                    issue_in(g + 2, p)
            return carry

        lax.fori_loop(0, NU // 2, pair_body, 0)
        wait_out(0)
        wait_out(1)

    return shear_kernel


def kernel(triangle01, triangle02):
    if "k" not in _cached:
        _cached["k"] = _build()
    return _cached["k"](triangle01, triangle02)


# speedup vs baseline: 3.7482x; 1.0161x over previous
import functools

import jax
import jax.numpy as jnp
from jax import lax
from jax.experimental import pallas as pl
from jax.experimental.pallas import tpu as pltpu
from jax.experimental.pallas import tpu_sc as plsc

P = 512
PAD = 16
W = P + 2 * PAD
B = 32
L = 16
NCHUNK = W // L
NU = P // L
SLABW = 128

_cached = {}


def _build():
    info = plsc.get_sparse_core_info()
    nc = info.num_cores
    mesh = plsc.VectorSubcoreMesh(core_axis_name="c", subcore_axis_name="s")

    scratch = [
        pltpu.VMEM((P + PAD, SLABW), jnp.float32),
        pltpu.VMEM((5 * L, SLABW), jnp.float32),
        pltpu.VMEM((5 * L, SLABW), jnp.float32),
        pltpu.VMEM((L, W), jnp.float32),
        pltpu.VMEM((L, W), jnp.float32),
    ] + [pltpu.SemaphoreType.DMA] * 5

    @functools.partial(
        pl.kernel,
        mesh=mesh,
        out_type=jax.ShapeDtypeStruct((B, P, W), jnp.float32),
        compiler_params=pltpu.CompilerParams(needs_layout_passes=False),
        scratch_types=scratch,
    )
    def shear_kernel(t01, t02, out, S, T0, T1, O0, O1, sS, sT0, sT1, sO0, sO1):
        b = lax.axis_index("s") * nc + lax.axis_index("c")
        iota = lax.iota(jnp.int32, L)
        zf = jnp.zeros((L,), jnp.float32)
        Tb, Ob = (T0, T1), (O0, O1)
        sT, sO = (sT0, sT1), (sO0, sO1)
        for rr in range(P, P + PAD):
            for cc8 in range(SLABW // L):
                S[rr, pl.ds(L * cc8, L)] = zf
        for Tp in Tb:
            for rr in range(4 * L, 5 * L):
                for cc8 in range(SLABW // L):
                    Tp[rr, pl.ds(L * cc8, L)] = zf

        slab_rows = [P, 400, 272, 144]

        def unit_i0(g):
            return (P - SLABW) - SLABW * (g >> 3) + L * (g & 7)

        def issue_slab(j):
            for s in range(4):
                @pl.when(j == 4 * s)
                def _(s=s):
                    nr = slab_rows[s]
                    pltpu.async_copy(
                        t02.at[b, pl.ds(0, nr), pl.ds(SLABW * s, SLABW)],
                        S.at[pl.ds(0, nr)], sS)
                    pltpu.make_async_copy(
                        t02.at[b, pl.ds(0, nr), pl.ds(0, SLABW)],
                        S.at[pl.ds(0, nr)], sS).wait()

        def issue_in(g, p):
            i0 = unit_i0(g)
            for tt in range(4):
                pltpu.async_copy(
                    t01.at[b, pl.ds(i0, L), pl.ds(SLABW * tt, SLABW)],
                    Tb[p].at[pl.ds(L * tt, L)], sT[p])

        def wait_in(p):
            for tt in range(4):
                pltpu.make_async_copy(
                    t01.at[b, pl.ds(0, L), pl.ds(SLABW * tt, SLABW)],
                    Tb[p].at[pl.ds(L * tt, L)], sT[p]).wait()

        def wait_out(p):
            pltpu.make_async_copy(Ob[p], out.at[b, pl.ds(0, L), :],
                                  sO[p]).wait()

        def compute_unit(g, p):
            i0 = unit_i0(g)
            q = i0 // L
            cslab = (P - 1) - i0 - SLABW * (g >> 3)
            T, O = Tb[p], Ob[p]

            def row_body(li, carry):
                i = i0 + li
                ccv = (cslab - li) + iota * 0
                liv = li + iota * 0

                def t2_four(_, c):
                    rv, om = c
                    for _ in range(4):
                        O[li, pl.ds(om, L)] = plsc.load_gather(S, [rv, ccv])
                        rv = rv - L
                        om = om + L
                    return rv, om

                na = q // 4
                rv0 = (PAD + i) - iota
                lax.fori_loop(0, na, t2_four, (rv0, 0))

                def gen_body(m, carry2):
                    sm = m * L
                    rm = (PAD + i - sm) - iota
                    cm = (sm - i - PAD - 1) + iota
                    gS = plsc.load_gather(S, [jnp.maximum(rm, 0), ccv])
                    cmc = jnp.maximum(cm, 0)
                    rowT = ((cmc >> 7) << 4) + liv
                    gT = plsc.load_gather(T, [rowT, cmc & (SLABW - 1)])
                    O[li, pl.ds(sm, L)] = jnp.where(rm >= 0, gS, gT)
                    return carry2

                lax.fori_loop(4 * na, q + 2, gen_body, 0)

                def t01_four(_, c):
                    cv, om = c
                    for _ in range(4):
                        rowT = ((cv >> 7) << 4) + liv
                        O[li, pl.ds(om, L)] = plsc.load_gather(
                            T, [rowT, cv & (SLABW - 1)])
                        cv = cv + L
                        om = om + L
                    return cv, om

                nb = (NU - q) // 4
                cend = lax.fori_loop(0, nb, t01_four,
                                     ((L - 1 - li) + iota, (q + 2) * L))

                def t01_one(m, carry2):
                    cm = (m * L - i - PAD - 1) + iota
                    rowT = ((cm >> 7) << 4) + liv
                    O[li, pl.ds(m * L, L)] = plsc.load_gather(
                        T, [rowT, cm & (SLABW - 1)])
                    return carry2

                lax.fori_loop(q + 2 + 4 * nb, NCHUNK, t01_one, 0)
                return carry

            lax.fori_loop(0, L, row_body, 0)
            pltpu.async_copy(O, out.at[b, pl.ds(i0, L), :], sO[p])

        issue_in(0, 0)
        issue_in(1, 1)

        def pair_body(j, carry):
            issue_slab(j)
            for p in (0, 1):
                g = 2 * j + p
                wait_in(p)

                @pl.when(j > 0)
                def _():
                    wait_out(p)

                compute_unit(g, p)

                @pl.when(j < NU // 2 - 1)
                def _():
                    issue_in(g + 2, p)
            return carry

        lax.fori_loop(0, NU // 2, pair_body, 0)
        wait_out(0)
        wait_out(1)

    return shear_kernel


def kernel(triangle01, triangle02):
    if "k" not in _cached:
        _cached["k"] = _build()
    return _cached["k"](triangle01, triangle02)


# speedup vs baseline: 3.9132x; 1.0440x over previous
import functools

import jax
import jax.numpy as jnp
from jax import lax
from jax.experimental import pallas as pl
from jax.experimental.pallas import tpu as pltpu
from jax.experimental.pallas import tpu_sc as plsc

P = 512
PAD = 16
W = P + 2 * PAD
B = 32
L = 16
NCHUNK = W // L
NU = P // L
SLABW = 128

_cached = {}


def _build():
    info = plsc.get_sparse_core_info()
    nc = info.num_cores
    mesh = plsc.VectorSubcoreMesh(core_axis_name="c", subcore_axis_name="s")

    scratch = [
        pltpu.VMEM((P + PAD, SLABW), jnp.float32),
        pltpu.VMEM((5 * L, SLABW), jnp.float32),
        pltpu.VMEM((5 * L, SLABW), jnp.float32),
        pltpu.VMEM((L + 1, W), jnp.float32),
        pltpu.VMEM((L + 1, W), jnp.float32),
    ] + [pltpu.SemaphoreType.DMA] * 5

    @functools.partial(
        pl.kernel,
        mesh=mesh,
        out_type=jax.ShapeDtypeStruct((B, P, W), jnp.float32),
        compiler_params=pltpu.CompilerParams(needs_layout_passes=False),
        scratch_types=scratch,
    )
    def shear_kernel(t01, t02, out, S, T0, T1, O0, O1, sS, sT0, sT1, sO0, sO1):
        b = lax.axis_index("s") * nc + lax.axis_index("c")
        iota = lax.iota(jnp.int32, L)
        zf = jnp.zeros((L,), jnp.float32)
        Tb, Ob = (T0, T1), (O0, O1)
        sT, sO = (sT0, sT1), (sO0, sO1)
        for rr in range(P, P + PAD):
            for cc8 in range(SLABW // L):
                S[rr, pl.ds(L * cc8, L)] = zf
        for Tp in Tb:
            for rr in range(4 * L, 5 * L):
                for cc8 in range(SLABW // L):
                    Tp[rr, pl.ds(L * cc8, L)] = zf

        slab_rows = [P, 400, 272, 144]

        def unit_i0(g):
            return (P - SLABW) - SLABW * (g >> 3) + L * (g & 7)

        def issue_slab(j):
            for s in range(4):
                @pl.when(j == 4 * s)
                def _(s=s):
                    nr = slab_rows[s]
                    pltpu.async_copy(
                        t02.at[b, pl.ds(0, nr), pl.ds(SLABW * s, SLABW)],
                        S.at[pl.ds(0, nr)], sS)
                    pltpu.make_async_copy(
                        t02.at[b, pl.ds(0, nr), pl.ds(0, SLABW)],
                        S.at[pl.ds(0, nr)], sS).wait()

        def issue_in(g, p):
            i0 = unit_i0(g)
            for tt in range(4):
                pltpu.async_copy(
                    t01.at[b, pl.ds(i0, L), pl.ds(SLABW * tt, SLABW)],
                    Tb[p].at[pl.ds(L * tt, L)], sT[p])

        def wait_in(p):
            for tt in range(4):
                pltpu.make_async_copy(
                    t01.at[b, pl.ds(0, L), pl.ds(SLABW * tt, SLABW)],
                    Tb[p].at[pl.ds(L * tt, L)], sT[p]).wait()

        def wait_out(p):
            pltpu.make_async_copy(Ob[p].at[pl.ds(0, L)],
                                  out.at[b, pl.ds(0, L), :], sO[p]).wait()

        def compute_unit(g, p):
            i0 = unit_i0(g)
            q = i0 // L
            cslab = (P - 1) - i0 - SLABW * (g >> 3)
            T, O = Tb[p], Ob[p]

            def row_body(li, carry):
                i = i0 + li
                ccv = (cslab - li) + iota * 0
                liv = li + iota * 0

                def t2_four(_, c):
                    rv, om = c
                    for _ in range(4):
                        O[li, pl.ds(om, L)] = plsc.load_gather(
                            S, [jnp.maximum(rv, 0), ccv])
                        rv = rv - L
                        om = om + L
                    return rv, om

                na = (q + 4) // 4
                lax.fori_loop(0, na, t2_four, ((PAD + i) - iota, 0))

                rm = liv - iota
                gS = plsc.load_gather(S, [jnp.maximum(rm, 0), ccv])
                gT = plsc.load_gather(T, [liv, jnp.maximum(iota - li - 1, 0)])
                O[li, pl.ds((q + 1) * L, L)] = jnp.where(rm >= 0, gS, gT)

                def t01_four(_, c):
                    cv, om = c
                    for _ in range(4):
                        rowT = ((cv >> 7) << 4) + liv
                        O[li, pl.ds(om, L)] = plsc.load_gather(
                            T, [rowT, cv & (SLABW - 1)])
                        cv = cv + L
                        om = om + L
                    return cv, om

                nb = (35 - q) // 4
                lax.fori_loop(0, nb, t01_four,
                              ((L - 1 - li) + iota, (q + 2) * L))
                return carry

            lax.fori_loop(0, L, row_body, 0)
            pltpu.async_copy(O.at[pl.ds(0, L)], out.at[b, pl.ds(i0, L), :],
                             sO[p])

        issue_in(0, 0)
        issue_in(1, 1)

        def pair_body(j, carry):
            issue_slab(j)
            for p in (0, 1):
                g = 2 * j + p
                wait_in(p)

                @pl.when(j > 0)
                def _():
                    wait_out(p)

                compute_unit(g, p)

                @pl.when(j < NU // 2 - 1)
                def _():
                    issue_in(g + 2, p)
            return carry

        lax.fori_loop(0, NU // 2, pair_body, 0)
        wait_out(0)
        wait_out(1)

    return shear_kernel


def kernel(triangle01, triangle02):
    if "k" not in _cached:
        _cached["k"] = _build()
    return _cached["k"](triangle01, triangle02)


# speedup vs baseline: 4.1019x; 1.0482x over previous
import functools

import jax
import jax.numpy as jnp
from jax import lax
from jax.experimental import pallas as pl
from jax.experimental.pallas import tpu as pltpu
from jax.experimental.pallas import tpu_sc as plsc

P = 512
PAD = 16
W = P + 2 * PAD
B = 32
L = 16
NCHUNK = W // L
NU = P // L
SLABW = 128

_cached = {}


def _build():
    info = plsc.get_sparse_core_info()
    nc = info.num_cores
    mesh = plsc.VectorSubcoreMesh(core_axis_name="c", subcore_axis_name="s")

    scratch = [
        pltpu.VMEM((P + PAD, SLABW), jnp.float32),
        pltpu.VMEM((5 * L, SLABW), jnp.float32),
        pltpu.VMEM((5 * L, SLABW), jnp.float32),
        pltpu.VMEM((L + 1, W), jnp.float32),
        pltpu.VMEM((L + 1, W), jnp.float32),
    ] + [pltpu.SemaphoreType.DMA] * 5

    @functools.partial(
        pl.kernel,
        mesh=mesh,
        out_type=jax.ShapeDtypeStruct((B, P, W), jnp.float32),
        compiler_params=pltpu.CompilerParams(needs_layout_passes=False),
        scratch_types=scratch,
    )
    def shear_kernel(t01, t02, out, S, T0, T1, O0, O1, sS, sT0, sT1, sO0, sO1):
        b = lax.axis_index("s") * nc + lax.axis_index("c")
        iota = lax.iota(jnp.int32, L)
        zf = jnp.zeros((L,), jnp.float32)
        Tb, Ob = (T0, T1), (O0, O1)
        sT, sO = (sT0, sT1), (sO0, sO1)
        for rr in range(P, P + PAD):
            for cc8 in range(SLABW // L):
                S[rr, pl.ds(L * cc8, L)] = zf
        for Tp in Tb:
            for rr in range(4 * L, 5 * L):
                for cc8 in range(SLABW // L):
                    Tp[rr, pl.ds(L * cc8, L)] = zf

        slab_rows = [P, 400, 272, 144]

        def unit_i0(g):
            return (P - SLABW) - SLABW * (g >> 3) + L * (g & 7)

        def issue_slab(j):
            for s in range(4):
                @pl.when(j == 4 * s)
                def _(s=s):
                    nr = slab_rows[s]
                    pltpu.async_copy(
                        t02.at[b, pl.ds(0, nr), pl.ds(SLABW * s, SLABW)],
                        S.at[pl.ds(0, nr)], sS)
                    pltpu.make_async_copy(
                        t02.at[b, pl.ds(0, nr), pl.ds(0, SLABW)],
                        S.at[pl.ds(0, nr)], sS).wait()

        def issue_in(g, p):
            i0 = unit_i0(g)
            for tt in range(4):
                pltpu.async_copy(
                    t01.at[b, pl.ds(i0, L), pl.ds(SLABW * tt, SLABW)],
                    Tb[p].at[pl.ds(L * tt, L)], sT[p])

        def wait_in(p):
            for tt in range(4):
                pltpu.make_async_copy(
                    t01.at[b, pl.ds(0, L), pl.ds(SLABW * tt, SLABW)],
                    Tb[p].at[pl.ds(L * tt, L)], sT[p]).wait()

        def wait_out(p):
            pltpu.make_async_copy(Ob[p].at[pl.ds(0, L)],
                                  out.at[b, pl.ds(0, L), :], sO[p]).wait()

        def compute_unit(g, p):
            i0 = unit_i0(g)
            q = i0 // L
            cslab = (P - 1) - i0 - SLABW * (g >> 3)
            T, O = Tb[p], Ob[p]

            def row_body(li):
                i = i0 + li
                ccv = (cslab - li) + iota * 0
                liv = li + iota * 0

                @plsc.parallel_loop(0, q + 1, unroll=8,
                                    carry=(PAD + i) - iota)
                def _(m, rv):
                    O[li, pl.ds(m * L, L)] = plsc.load_gather(
                        S, [jnp.maximum(rv, 0), ccv])
                    return rv - L

                rm = liv - iota
                gS = plsc.load_gather(S, [jnp.maximum(rm, 0), ccv])
                gT = plsc.load_gather(T, [liv, jnp.maximum(iota - li - 1, 0)])
                O[li, pl.ds((q + 1) * L, L)] = jnp.where(rm >= 0, gS, gT)

                cv0 = (L - 1 - li - (q + 2) * L) + iota

                @plsc.parallel_loop(q + 2, NCHUNK, unroll=8)
                def _(m):
                    cv = cv0 + m * L
                    rowT = ((cv >> 7) << 4) + liv
                    O[li, pl.ds(m * L, L)] = plsc.load_gather(
                        T, [rowT, cv & (SLABW - 1)])

            plsc.parallel_loop(0, L)(row_body)
            pltpu.async_copy(O.at[pl.ds(0, L)], out.at[b, pl.ds(i0, L), :],
                             sO[p])

        issue_in(0, 0)
        issue_in(1, 1)

        def pair_body(j, carry):
            issue_slab(j)
            for p in (0, 1):
                g = 2 * j + p
                wait_in(p)

                @pl.when(j > 0)
                def _():
                    wait_out(p)

                compute_unit(g, p)

                @pl.when(j < NU // 2 - 1)
                def _():
                    issue_in(g + 2, p)
            return carry

        lax.fori_loop(0, NU // 2, pair_body, 0)
        wait_out(0)
        wait_out(1)

    return shear_kernel


def kernel(triangle01, triangle02):
    if "k" not in _cached:
        _cached["k"] = _build()
    return _cached["k"](triangle01, triangle02)
